# baseline (speedup 1.0000x reference)
import functools

import jax
import jax.numpy as jnp
from jax import lax
from jax.experimental import pallas as pl
from jax.experimental.pallas import tpu as pltpu
from jax.experimental.pallas import tpu_sc as plsc

_N = 100000
_K = 9
_DF = 32
_DS = 16
_L = 16
_NC = 2
_NS = 16
_NW = _NC * _NS
_CH = 64
_PPW = 3200
_NPAD = _NW * _PPW
_NCH = _PPW // _CH
_NBUF = 2

_WF = 1.0 / (_N * _DF)
_W3 = 1.0 / (_N * 3)
_W1 = 1.0 / _N


def _body(feat, small, idxt, out, idx_v, rf_v, rs_v, out_v, s0, s1):
    sems = (s0, s1)
    wid = lax.axis_index("s") * _NC + lax.axis_index("c")

    for j in range(_K):
        pltpu.sync_copy(idxt.at[j, pl.ds(wid * _PPW, _PPW)], idx_v.at[j])

    io = lax.iota(jnp.int32, _L)
    wsm = jnp.where(io < 6, jnp.float32(_W3),
                    jnp.where(io == 6, jnp.float32(_W1),
                              jnp.zeros((_L,), jnp.float32)))

    def fire(c, b):
        for j in range(_K):
            ix = idx_v.at[j, pl.ds(c * _CH, _CH)]
            pltpu.async_copy(feat.at[ix], rf_v.at[b, j], sems[b])
            pltpu.async_copy(small.at[ix], rs_v.at[b, j], sems[b])

    def drain(b):
        for j in range(_K):
            ix = idx_v.at[j, pl.ds(0, _CH)]
            pltpu.make_async_copy(feat.at[ix], rf_v.at[b, j],
                                  sems[b]).wait()
            pltpu.make_async_copy(small.at[ix], rs_v.at[b, j],
                                  sems[b]).wait()

    for b in range(_NBUF):
        fire(b, b)

    def outer(c2, acc):
        c = c2 * _NBUF
        for b in range(_NBUF):
            drain(b)

            def point(p, a, b=b):
                for g in range(2):
                    col = pl.ds(g * _L, _L)
                    cv = rf_v[b, 0, p, col]
                    s = rf_v[b, 1, p, col]
                    for j in range(2, _K):
                        s = s + rf_v[b, j, p, col]
                    d = cv - 0.125 * s
                    a = a + (d * d) * _WF
                cs = rs_v[b, 0, p, :]
                ss = rs_v[b, 1, p, :]
                for j in range(2, _K):
                    ss = ss + rs_v[b, j, p, :]
                d = cs - 0.125 * ss
                return a + (d * d) * wsm

            acc = lax.fori_loop(0, _CH, point, acc)

            cn = c + b + _NBUF

            @pl.when(cn < _NCH)
            def _(b=b, cn=cn):
                fire(cn, b)

        return acc

    acc = lax.fori_loop(0, _NCH // _NBUF, outer,
                        jnp.zeros((_L,), jnp.float32))
    out_v[...] = acc
    pltpu.sync_copy(out_v, out.at[pl.ds(wid * _L, _L)])


_kern = functools.partial(
    pl.kernel,
    out_type=jax.ShapeDtypeStruct((_NW * _L,), jnp.float32),
    mesh=plsc.VectorSubcoreMesh(core_axis_name="c", subcore_axis_name="s"),
    scratch_types=[
        pltpu.VMEM((_K, _PPW), jnp.int32),
        pltpu.VMEM((_NBUF, _K, _CH, _DF), jnp.float32),
        pltpu.VMEM((_NBUF, _K, _CH, _DS), jnp.float32),
        pltpu.VMEM((_L,), jnp.float32),
        pltpu.SemaphoreType.DMA,
        pltpu.SemaphoreType.DMA,
    ],
    compiler_params=pltpu.CompilerParams(use_tc_tiling_on_sc=False),
)(_body)


def kernel(xyz_dis, scaling, opacity, neural_features, lap_indices):
    n = xyz_dis.shape[0]
    small = jnp.concatenate(
        [xyz_dis, scaling, opacity, jnp.zeros((n, _DS - 7), jnp.float32)],
        axis=1)
    idxt = jnp.pad(lap_indices.astype(jnp.int32).T,
                   ((0, 0), (0, _NPAD - n)))
    parts = _kern(neural_features, small, idxt)
    return jnp.sum(parts)


# speedup vs baseline: 1.5413x; 1.5413x over previous
import functools

import jax
import jax.numpy as jnp
from jax import lax
from jax.experimental import pallas as pl
from jax.experimental.pallas import tpu as pltpu
from jax.experimental.pallas import tpu_sc as plsc

_N = 100000
_K = 9
_DF = 32
_DS = 16
_L = 16
_NC = 2
_NS = 16
_NW = _NC * _NS
_CP = 8
_IPT = _CP * _K
_NBUF = 4

_PPW = 3136
_NPAD = _NW * _PPW
_TPW = _PPW // _CP
_IPW = _PPW * _K

_WF = 1.0 / (_N * _DF)
_W3 = 1.0 / (_N * 3)
_W1 = 1.0 / _N


def _body(feat, small, idxh, out, idx_v, rf_v, rs_v, out_v, s0, s1, s2, s3):
    sems = (s0, s1, s2, s3)
    wid = lax.axis_index("s") * _NC + lax.axis_index("c")

    pltpu.sync_copy(idxh.at[pl.ds(wid * _IPW, _IPW)], idx_v)

    io = lax.iota(jnp.int32, _L)
    wsm = jnp.where(io < 6, jnp.float32(_W3),
                    jnp.where(io == 6, jnp.float32(_W1),
                              jnp.zeros((_L,), jnp.float32)))

    def fire(t, b):
        ix = idx_v.at[pl.ds(t * _IPT, _IPT)]
        pltpu.async_copy(feat.at[ix], rf_v.at[b], sems[b])
        pltpu.async_copy(small.at[ix], rs_v.at[b], sems[b])

    for b in range(_NBUF):
        fire(b, b)

    def outer(t2, acc):
        t = t2 * _NBUF
        for b in range(_NBUF):
            ixw = idx_v.at[pl.ds(b * _IPT, _IPT)]
            pltpu.make_async_copy(feat.at[ixw], rf_v.at[b], sems[b]).wait()
            pltpu.make_async_copy(small.at[ixw], rs_v.at[b], sems[b]).wait()

            def point(p, a, b=b):
                r = p * _K
                for g in range(2):
                    col = pl.ds(g * _L, _L)
                    cv = rf_v[b, r, col]
                    s = rf_v[b, r + 1, col]
                    for j in range(2, _K):
                        s = s + rf_v[b, r + j, col]
                    d = cv - 0.125 * s
                    a = a + (d * d) * _WF
                cs = rs_v[b, r, :]
                ss = rs_v[b, r + 1, :]
                for j in range(2, _K):
                    ss = ss + rs_v[b, r + j, :]
                d = cs - 0.125 * ss
                return a + (d * d) * wsm

            acc = lax.fori_loop(0, _CP, point, acc)

            tn = t + b + _NBUF

            @pl.when(tn < _TPW)
            def _(b=b, tn=tn):
                fire(tn, b)

        return acc

    acc = lax.fori_loop(0, _TPW // _NBUF, outer,
                        jnp.zeros((_L,), jnp.float32))
    out_v[...] = acc
    pltpu.sync_copy(out_v, out.at[pl.ds(wid * _L, _L)])


_kern = functools.partial(
    pl.kernel,
    out_type=jax.ShapeDtypeStruct((_NW * _L,), jnp.float32),
    mesh=plsc.VectorSubcoreMesh(core_axis_name="c", subcore_axis_name="s"),
    scratch_types=[
        pltpu.VMEM((_IPW,), jnp.int32),
        pltpu.VMEM((_NBUF, _IPT, _DF), jnp.float32),
        pltpu.VMEM((_NBUF, _IPT, _DS), jnp.float32),
        pltpu.VMEM((_L,), jnp.float32),
        pltpu.SemaphoreType.DMA,
        pltpu.SemaphoreType.DMA,
        pltpu.SemaphoreType.DMA,
        pltpu.SemaphoreType.DMA,
    ],
    compiler_params=pltpu.CompilerParams(use_tc_tiling_on_sc=False),
)(_body)


def kernel(xyz_dis, scaling, opacity, neural_features, lap_indices):
    n = xyz_dis.shape[0]
    small = jnp.concatenate(
        [xyz_dis, scaling, opacity, jnp.zeros((n, _DS - 7), jnp.float32)],
        axis=1)
    idxflat = jnp.pad(lap_indices.astype(jnp.int32).reshape(-1),
                      (0, (_NPAD - n) * _K))
    parts = _kern(neural_features, small, idxflat)
    return jnp.sum(parts)


# speedup vs baseline: 1.6923x; 1.0980x over previous
import functools

import jax
import jax.numpy as jnp
from jax import lax
from jax.experimental import pallas as pl
from jax.experimental.pallas import tpu as pltpu
from jax.experimental.pallas import tpu_sc as plsc

_N = 100000
_K = 9
_DF = 32
_DS = 16
_L = 16
_NC = 2
_NS = 16
_NW = _NC * _NS
_CP = 8
_IPT = _CP * _K
_NBUF = 8

_PPW = 3136
_IPW = _PPW * _K
_NPAD = _NW * _PPW
_TPW = _PPW // _CP

_WF = 1.0 / (_N * _DF)
_W3 = 1.0 / (_N * 3)
_W1 = 1.0 / _N


def _body(feat, small, idxh, out, idx_v, rf_v, rs_v, out_v,
          s0, s1, s2, s3, s4, s5, s6, s7):
    sems = (s0, s1, s2, s3, s4, s5, s6, s7)
    wid = lax.axis_index("s") * _NC + lax.axis_index("c")

    io = lax.iota(jnp.int32, _L)
    pltpu.sync_copy(idxh.at[pl.ds(wid * _IPW, _IPW)], idx_v)

    wsm = jnp.where(io < 6, jnp.float32(_W3),
                    jnp.where(io == 6, jnp.float32(_W1),
                              jnp.zeros((_L,), jnp.float32)))

    def fire(t, b):
        ix = idx_v.at[pl.ds(t * _IPT, _IPT)]
        pltpu.async_copy(feat.at[ix], rf_v.at[b], sems[b])
        pltpu.async_copy(small.at[ix], rs_v.at[b], sems[b])

    for b in range(_NBUF):
        fire(b, b)

    def outer(t2, acc):
        t = t2 * _NBUF
        for b in range(_NBUF):
            ixw = idx_v.at[pl.ds(b * _IPT, _IPT)]
            pltpu.make_async_copy(feat.at[ixw], rf_v.at[b], sems[b]).wait()
            pltpu.make_async_copy(small.at[ixw], rs_v.at[b], sems[b]).wait()

            def point(p, a, b=b):
                r = p * _K
                for g in range(2):
                    col = pl.ds(g * _L, _L)
                    cv = rf_v[b, r, col]
                    s = rf_v[b, r + 1, col]
                    for j in range(2, _K):
                        s = s + rf_v[b, r + j, col]
                    d = cv - 0.125 * s
                    a = a + (d * d) * _WF
                cs = rs_v[b, r, :]
                ss = rs_v[b, r + 1, :]
                for j in range(2, _K):
                    ss = ss + rs_v[b, r + j, :]
                d = cs - 0.125 * ss
                return a + (d * d) * wsm

            acc = lax.fori_loop(0, _CP, point, acc)

            tn = t + b + _NBUF

            @pl.when(tn < _TPW)
            def _(b=b, tn=tn):
                fire(tn, b)

        return acc

    acc = lax.fori_loop(0, _TPW // _NBUF, outer,
                        jnp.zeros((_L,), jnp.float32))
    out_v[...] = acc
    pltpu.sync_copy(out_v, out.at[pl.ds(wid * _L, _L)])


_kern = functools.partial(
    pl.kernel,
    out_type=jax.ShapeDtypeStruct((_NW * _L,), jnp.float32),
    mesh=plsc.VectorSubcoreMesh(core_axis_name="c", subcore_axis_name="s"),
    scratch_types=[
        pltpu.VMEM((_IPW,), jnp.int32),
        pltpu.VMEM((_NBUF, _IPT, _DF), jnp.float32),
        pltpu.VMEM((_NBUF, _IPT, _DS), jnp.float32),
        pltpu.VMEM((_L,), jnp.float32),
        pltpu.SemaphoreType.DMA,
        pltpu.SemaphoreType.DMA,
        pltpu.SemaphoreType.DMA,
        pltpu.SemaphoreType.DMA,
        pltpu.SemaphoreType.DMA,
        pltpu.SemaphoreType.DMA,
        pltpu.SemaphoreType.DMA,
        pltpu.SemaphoreType.DMA,
    ],
    compiler_params=pltpu.CompilerParams(use_tc_tiling_on_sc=False),
)(_body)


def kernel(xyz_dis, scaling, opacity, neural_features, lap_indices):
    n = xyz_dis.shape[0]
    small = jnp.concatenate(
        [xyz_dis, scaling, opacity, jnp.zeros((n, _DS - 7), jnp.float32)],
        axis=1)
    idxflat = jnp.pad(lap_indices.astype(jnp.int32).reshape(-1),
                      (0, (_NPAD - n) * _K))
    parts = _kern(neural_features, small, idxflat)
    return jnp.sum(parts)
